# baseline (device time: 293648 ns/iter reference)
import jax
import jax.numpy as jnp
from jax import lax
from jax.experimental import pallas as pl
from jax.experimental.pallas import tpu as pltpu

N_DEV = 4
SUB = 256
C = 2
N_SLOTS = C + 2
N_STEPS = (N_DEV - 1) * C


def kernel(x, w_mat):
    m_per, k = x.shape
    _, n_per = w_mat.shape

    def body(x_ref, w_ref, out_ref, cw_ref, ccw_ref,
             cw_send, cw_recv, ccw_send, ccw_recv, cw_credit, ccw_credit,
             own_stage, stage, own_osem, osems):
        me = lax.axis_index("i")
        right = lax.rem(me + 1, N_DEV)
        left = lax.rem(me + N_DEV - 1, N_DEV)

        def silu_gemm(a):
            y = jnp.dot(a, w_ref[:, :], preferred_element_type=jnp.float32)
            return y * jax.nn.sigmoid(y)

        bufs = (cw_ref, ccw_ref)
        send_sems = (cw_send, ccw_send)
        recv_sems = (cw_recv, ccw_recv)
        credits = (cw_credit, ccw_credit)
        dst_dev = (right, left)
        upstream = (left, right)
        half_off = (0, m_per // 2)

        barrier_sem = pltpu.get_barrier_semaphore()
        for nbr in (left, right):
            pl.semaphore_signal(
                barrier_sem, inc=1,
                device_id=(nbr,), device_id_type=pl.DeviceIdType.MESH,
            )
        pl.semaphore_wait(barrier_sem, 2)

        def make_rdma(s, d):
            if s < C:
                lo = half_off[d] + s * SUB
                src = x_ref.at[lo:lo + SUB]
            else:
                src = bufs[d].at[(s - C) % N_SLOTS]
            return pltpu.make_async_remote_copy(
                src_ref=src,
                dst_ref=bufs[d].at[s % N_SLOTS],
                send_sem=send_sems[d].at[s % N_SLOTS],
                recv_sem=recv_sems[d].at[s % N_SLOTS],
                device_id=(dst_dev[d],),
                device_id_type=pl.DeviceIdType.MESH,
            )

        def chunk_rows(s, d):
            hop = s // C + 1
            if d == 0:
                origin = lax.rem(me + 2 * N_DEV - hop, N_DEV)
            else:
                origin = lax.rem(me + hop, N_DEV)
            return origin * m_per + half_off[d] + (s % C) * SUB

        rd = {}
        for d in (0, 1):
            for s in range(C):
                rd[(s, d)] = make_rdma(s, d)
                rd[(s, d)].start()

        own_stage[:, :] = silu_gemm(x_ref[:, :])
        own_dma = pltpu.make_async_copy(
            own_stage, out_ref.at[pl.ds(me * m_per, m_per), :], own_osem)
        own_dma.start()

        odma = {}
        for s in range(N_STEPS):
            for d in (0, 1):
                rd[(s, d)].wait_recv()
                rd[(s, d)].wait_send()
                if C <= s < C + (N_STEPS - N_SLOTS):
                    pl.semaphore_signal(
                        credits[d], inc=1,
                        device_id=(upstream[d],),
                        device_id_type=pl.DeviceIdType.MESH,
                    )
                if s < N_STEPS - C:
                    if s >= N_SLOTS - C:
                        pl.semaphore_wait(credits[d], 1)
                    rd[(s + C, d)] = make_rdma(s + C, d)
                    rd[(s + C, d)].start()
            for d in (0, 1):
                if s >= 2:
                    odma[(s - 2, d)].wait()
                stage[d, s % 2, :, :] = silu_gemm(bufs[d][s % N_SLOTS, :, :])
                odma[(s, d)] = pltpu.make_async_copy(
                    stage.at[d, s % 2],
                    out_ref.at[pl.ds(chunk_rows(s, d), SUB), :],
                    osems.at[d, s % 2],
                )
                odma[(s, d)].start()

        own_dma.wait()
        for d in (0, 1):
            odma[(N_STEPS - 2, d)].wait()
            odma[(N_STEPS - 1, d)].wait()

    return pl.pallas_call(
        body,
        out_shape=jax.ShapeDtypeStruct((N_DEV * m_per, n_per), jnp.float32),
        in_specs=[
            pl.BlockSpec(memory_space=pltpu.VMEM),
            pl.BlockSpec(memory_space=pltpu.VMEM),
        ],
        out_specs=pl.BlockSpec(memory_space=pl.ANY),
        scratch_shapes=[
            pltpu.VMEM((N_SLOTS, SUB, k), jnp.float32),
            pltpu.VMEM((N_SLOTS, SUB, k), jnp.float32),
            pltpu.SemaphoreType.DMA((N_SLOTS,)),
            pltpu.SemaphoreType.DMA((N_SLOTS,)),
            pltpu.SemaphoreType.DMA((N_SLOTS,)),
            pltpu.SemaphoreType.DMA((N_SLOTS,)),
            pltpu.SemaphoreType.REGULAR,
            pltpu.SemaphoreType.REGULAR,
            pltpu.VMEM((m_per, n_per), jnp.float32),
            pltpu.VMEM((2, 2, SUB, n_per), jnp.float32),
            pltpu.SemaphoreType.DMA,
            pltpu.SemaphoreType.DMA((2, 2)),
        ],
        compiler_params=pltpu.CompilerParams(
            collective_id=0,
            vmem_limit_bytes=62 * 1024 * 1024,
        ),
    )(x, w_mat)


# device time: 292391 ns/iter; 1.0043x vs baseline; 1.0043x over previous
import jax
import jax.numpy as jnp
from jax import lax
from jax.experimental import pallas as pl
from jax.experimental.pallas import tpu as pltpu

N_DEV = 4
SUB = 128
C = 4
N_SLOTS = C + 4
N_STEPS = (N_DEV - 1) * C


def kernel(x, w_mat):
    m_per, k = x.shape
    _, n_per = w_mat.shape

    def body(x_ref, w_ref, out_ref, cw_ref, ccw_ref,
             cw_send, cw_recv, ccw_send, ccw_recv, cw_credit, ccw_credit,
             own_stage, stage, own_osem, osems):
        me = lax.axis_index("i")
        right = lax.rem(me + 1, N_DEV)
        left = lax.rem(me + N_DEV - 1, N_DEV)

        def silu_gemm(a):
            y = jnp.dot(a, w_ref[:, :], preferred_element_type=jnp.float32)
            return y * jax.nn.sigmoid(y)

        bufs = (cw_ref, ccw_ref)
        send_sems = (cw_send, ccw_send)
        recv_sems = (cw_recv, ccw_recv)
        credits = (cw_credit, ccw_credit)
        dst_dev = (right, left)
        upstream = (left, right)
        half_off = (0, m_per // 2)

        barrier_sem = pltpu.get_barrier_semaphore()
        for nbr in (left, right):
            pl.semaphore_signal(
                barrier_sem, inc=1,
                device_id=(nbr,), device_id_type=pl.DeviceIdType.MESH,
            )
        pl.semaphore_wait(barrier_sem, 2)

        def make_rdma(s, d):
            if s < C:
                lo = half_off[d] + s * SUB
                src = x_ref.at[lo:lo + SUB]
            else:
                src = bufs[d].at[(s - C) % N_SLOTS]
            return pltpu.make_async_remote_copy(
                src_ref=src,
                dst_ref=bufs[d].at[s % N_SLOTS],
                send_sem=send_sems[d].at[s % N_SLOTS],
                recv_sem=recv_sems[d].at[s % N_SLOTS],
                device_id=(dst_dev[d],),
                device_id_type=pl.DeviceIdType.MESH,
            )

        def chunk_rows(s, d):
            hop = s // C + 1
            if d == 0:
                origin = lax.rem(me + 2 * N_DEV - hop, N_DEV)
            else:
                origin = lax.rem(me + hop, N_DEV)
            return origin * m_per + half_off[d] + (s % C) * SUB

        rd = {}
        for d in (0, 1):
            for s in range(C):
                rd[(s, d)] = make_rdma(s, d)
                rd[(s, d)].start()

        own_stage[:, :] = silu_gemm(x_ref[:, :])
        own_dma = pltpu.make_async_copy(
            own_stage, out_ref.at[pl.ds(me * m_per, m_per), :], own_osem)
        own_dma.start()

        odma = {}
        for s in range(N_STEPS):
            for d in (0, 1):
                rd[(s, d)].wait_recv()
                rd[(s, d)].wait_send()
                if C <= s < C + (N_STEPS - N_SLOTS):
                    pl.semaphore_signal(
                        credits[d], inc=1,
                        device_id=(upstream[d],),
                        device_id_type=pl.DeviceIdType.MESH,
                    )
                if s < N_STEPS - C:
                    if s >= N_SLOTS - C:
                        pl.semaphore_wait(credits[d], 1)
                    rd[(s + C, d)] = make_rdma(s + C, d)
                    rd[(s + C, d)].start()
            for d in (0, 1):
                if s >= 2:
                    odma[(s - 2, d)].wait()
                stage[d, s % 2, :, :] = silu_gemm(bufs[d][s % N_SLOTS, :, :])
                odma[(s, d)] = pltpu.make_async_copy(
                    stage.at[d, s % 2],
                    out_ref.at[pl.ds(chunk_rows(s, d), SUB), :],
                    osems.at[d, s % 2],
                )
                odma[(s, d)].start()

        own_dma.wait()
        for d in (0, 1):
            odma[(N_STEPS - 2, d)].wait()
            odma[(N_STEPS - 1, d)].wait()

    return pl.pallas_call(
        body,
        out_shape=jax.ShapeDtypeStruct((N_DEV * m_per, n_per), jnp.float32),
        in_specs=[
            pl.BlockSpec(memory_space=pltpu.VMEM),
            pl.BlockSpec(memory_space=pltpu.VMEM),
        ],
        out_specs=pl.BlockSpec(memory_space=pl.ANY),
        scratch_shapes=[
            pltpu.VMEM((N_SLOTS, SUB, k), jnp.float32),
            pltpu.VMEM((N_SLOTS, SUB, k), jnp.float32),
            pltpu.SemaphoreType.DMA((N_SLOTS,)),
            pltpu.SemaphoreType.DMA((N_SLOTS,)),
            pltpu.SemaphoreType.DMA((N_SLOTS,)),
            pltpu.SemaphoreType.DMA((N_SLOTS,)),
            pltpu.SemaphoreType.REGULAR,
            pltpu.SemaphoreType.REGULAR,
            pltpu.VMEM((m_per, n_per), jnp.float32),
            pltpu.VMEM((2, 2, SUB, n_per), jnp.float32),
            pltpu.SemaphoreType.DMA,
            pltpu.SemaphoreType.DMA((2, 2)),
        ],
        compiler_params=pltpu.CompilerParams(
            collective_id=0,
            vmem_limit_bytes=62 * 1024 * 1024,
        ),
    )(x, w_mat)
